# baseline (speedup 1.0000x reference)
import jax
import jax.numpy as jnp
from jax import lax
from jax.experimental import pallas as pl
from jax.experimental.pallas import tpu as pltpu
from jax.experimental.pallas import tpu_sc as plsc

BATCH = 1024
TAU = 30.0
NCLS = 10
NC, NS, L = 2, 16, 16
NW = NC * NS
OUT_PAD = 8192
BPW = OUT_PAD // NW
GRP = 16
NGRP = BPW // GRP

_GATE_K = (
    (0, 0, 0, 0), (0, 0, 0, 1), (0, 1, 0, -1), (0, 1, 0, 0),
    (0, 0, 1, -1), (0, 0, 1, 0), (0, 1, 1, -2), (0, 1, 1, -1),
    (1, -1, -1, 1), (1, -1, -1, 2), (1, 0, -1, 0), (1, 0, -1, 1),
    (1, -1, 0, 0), (1, -1, 0, 1), (1, 0, 0, -1), (1, 0, 0, 0),
)


def _coef_tc(wall):
    def body(w_ref, k_ref, o_ref):
        w = w_ref[...]
        m = jnp.max(w, axis=-1, keepdims=True)
        e = jnp.exp(w - m)
        p = e / jnp.sum(e, axis=-1, keepdims=True)
        o_ref[...] = jax.lax.dot(p, k_ref[...], precision=lax.Precision.HIGHEST)

    n = wall.shape[0]
    blk = 2048
    return pl.pallas_call(
        body,
        grid=(n // blk,),
        in_specs=[
            pl.BlockSpec((blk, 16), lambda i: (i, 0)),
            pl.BlockSpec((16, 4), lambda i: (0, 0)),
        ],
        out_specs=pl.BlockSpec((blk, 4), lambda i: (i, 0)),
        out_shape=jax.ShapeDtypeStruct((n, 4), jnp.float32),
    )(wall, jnp.asarray(_GATE_K, dtype=jnp.float32))


def _sc_layer(table, aidx, bidx, cfs):
    mesh = plsc.VectorSubcoreMesh(core_axis_name="c", subcore_axis_name="s")

    def body(tab, ai, bi, cf, out, aiv, biv, cfv,
             abufs, bbufs, obufs, sems_a, sems_b, sems_o):
        wid = lax.axis_index("s") * NC + lax.axis_index("c")
        base = wid * BPW
        pltpu.sync_copy(ai.at[pl.ds(base, BPW)], aiv)
        pltpu.sync_copy(bi.at[pl.ds(base, BPW)], biv)
        pltpu.sync_copy(cf.at[pl.ds(base * 4 * L, BPW * 4 * L)], cfv)

        def issue(g):
            s = g % 2
            r0 = g * GRP
            cpa = pltpu.async_copy(
                tab.at[aiv.at[pl.ds(r0, GRP)]], abufs[s], sems_a[s])
            cpb = pltpu.async_copy(
                tab.at[biv.at[pl.ds(r0, GRP)]], bbufs[s], sems_b[s])
            return cpa, cpb

        pend = {0: issue(0)}
        out_pend = {}
        for g in range(NGRP):
            s = g % 2
            if g + 1 < NGRP:
                pend[g + 1] = issue(g + 1)
            cpa, cpb = pend.pop(g)
            cpa.wait()
            cpb.wait()
            if g >= 2:
                out_pend.pop(g - 2).wait()
            abuf, bbuf, obuf = abufs[s], bbufs[s], obufs[s]
            r0 = g * GRP

            def row_fn(r, carry, abuf=abuf, bbuf=bbuf, obuf=obuf, r0=r0):
                off = (r0 + r) * 4 * L
                c0 = cfv[pl.ds(off, L)]
                c1 = cfv[pl.ds(off + L, L)]
                c2 = cfv[pl.ds(off + 2 * L, L)]
                c3 = cfv[pl.ds(off + 3 * L, L)]

                def col_fn(j, carry2):
                    sl = pl.ds(j * L, L)
                    av = abuf[r, sl]
                    bv = bbuf[r, sl]
                    obuf[r, sl] = (c0 + c1 * av) + (c2 + c3 * av) * bv
                    return carry2

                lax.fori_loop(0, BATCH // L, col_fn, 0, unroll=8)
                return carry

            lax.fori_loop(0, GRP, row_fn, 0)
            out_pend[g] = pltpu.async_copy(
                obuf, out.at[pl.ds(base + r0, GRP)], sems_o[s])
        for g in sorted(out_pend):
            out_pend.pop(g).wait()

    kfn = pl.kernel(
        body,
        out_type=jax.ShapeDtypeStruct((OUT_PAD, BATCH), jnp.float32),
        mesh=mesh,
        scratch_types=[
            pltpu.VMEM((BPW,), jnp.int32),
            pltpu.VMEM((BPW,), jnp.int32),
            pltpu.VMEM((BPW * 4 * L,), jnp.float32),
            [pltpu.VMEM((GRP, BATCH), jnp.float32)] * 2,
            [pltpu.VMEM((GRP, BATCH), jnp.float32)] * 2,
            [pltpu.VMEM((GRP, BATCH), jnp.float32)] * 2,
            [pltpu.SemaphoreType.DMA] * 2,
            [pltpu.SemaphoreType.DMA] * 2,
            [pltpu.SemaphoreType.DMA] * 2,
        ],
    )
    return kfn(table, aidx, bidx, cfs)


def _gsum_tc(y, n_valid):
    rows = n_valid // NCLS

    def body(y_ref, o_ref):
        o_ref[...] = (jnp.sum(y_ref[...], axis=0, keepdims=True) / TAU)[None]

    return pl.pallas_call(
        body,
        grid=(NCLS,),
        in_specs=[pl.BlockSpec((rows, BATCH), lambda c: (c, 0))],
        out_specs=pl.BlockSpec((1, 1, BATCH), lambda c: (c, 0, 0)),
        out_shape=jax.ShapeDtypeStruct((NCLS, 1, BATCH), jnp.float32),
    )(y)


def kernel(x, w1, w2, w3, a1, b1, a2, b2, a3, b3):
    xt = x.T
    n3 = w3.shape[0]
    w3p = jnp.concatenate([w3, jnp.zeros((OUT_PAD - n3, 16), jnp.float32)], 0)
    wall = jnp.concatenate([w1, w2, w3p], axis=0)
    coefs = _coef_tc(wall)
    cfs = jnp.broadcast_to(coefs[:, :, None], (coefs.shape[0], 4, L))
    cf1 = cfs[:OUT_PAD].reshape(-1)
    cf2 = cfs[OUT_PAD:2 * OUT_PAD].reshape(-1)
    cf3 = cfs[2 * OUT_PAD:].reshape(-1)
    padi = jnp.zeros((OUT_PAD - n3,), jnp.int32)
    a3p = jnp.concatenate([a3, padi])
    b3p = jnp.concatenate([b3, padi])
    y1 = _sc_layer(xt, a1, b1, cf1)
    y2 = _sc_layer(y1, a2, b2, cf2)
    y3 = _sc_layer(y2, a3p, b3p, cf3)
    cls = _gsum_tc(y3, n3)
    return cls.reshape(NCLS, BATCH).T


# speedup vs baseline: 1.7593x; 1.7593x over previous
import jax
import jax.numpy as jnp
from jax import lax
from jax.experimental import pallas as pl
from jax.experimental.pallas import tpu as pltpu
from jax.experimental.pallas import tpu_sc as plsc

BATCH = 1024
TAU = 30.0
NCLS = 10
NC, NS, L = 2, 16, 16
NW = NC * NS
OUT_PAD = 8192
BPW = OUT_PAD // NW
GRP = 16
NGRP = BPW // GRP

_GATE_K = (
    (0, 0, 0, 0), (0, 0, 0, 1), (0, 1, 0, -1), (0, 1, 0, 0),
    (0, 0, 1, -1), (0, 0, 1, 0), (0, 1, 1, -2), (0, 1, 1, -1),
    (1, -1, -1, 1), (1, -1, -1, 2), (1, 0, -1, 0), (1, 0, -1, 1),
    (1, -1, 0, 0), (1, -1, 0, 1), (1, 0, 0, -1), (1, 0, 0, 0),
)


def _coef_tc(wall):
    def body(w_ref, k_ref, o_ref):
        w = w_ref[...]
        m = jnp.max(w, axis=-1, keepdims=True)
        e = jnp.exp(w - m)
        p = e / jnp.sum(e, axis=-1, keepdims=True)
        o_ref[...] = jax.lax.dot(p, k_ref[...], precision=lax.Precision.HIGHEST)

    n = wall.shape[0]
    blk = 2048
    return pl.pallas_call(
        body,
        grid=(n // blk,),
        in_specs=[
            pl.BlockSpec((blk, 16), lambda i: (i, 0)),
            pl.BlockSpec((16, 4), lambda i: (0, 0)),
        ],
        out_specs=pl.BlockSpec((blk, 4), lambda i: (i, 0)),
        out_shape=jax.ShapeDtypeStruct((n, 4), jnp.float32),
    )(wall, jnp.asarray(_GATE_K, dtype=jnp.float32))


def _sc_layer(table, aidx, bidx, cfs):
    mesh = plsc.VectorSubcoreMesh(core_axis_name="c", subcore_axis_name="s")

    def body(tab, ai, bi, cf, out, aiv, biv, cfv,
             abufs, bbufs, obufs, sems_a, sems_b, sems_o):
        wid = lax.axis_index("s") * NC + lax.axis_index("c")
        base = wid * BPW
        pltpu.sync_copy(ai.at[pl.ds(base, BPW)], aiv)
        pltpu.sync_copy(bi.at[pl.ds(base, BPW)], biv)
        pltpu.sync_copy(cf.at[:, pl.ds(base, BPW)], cfv)

        def issue(g):
            s = g % 2
            r0 = g * GRP
            cpa = pltpu.async_copy(
                tab.at[aiv.at[pl.ds(r0, GRP)]], abufs[s], sems_a[s])
            cpb = pltpu.async_copy(
                tab.at[biv.at[pl.ds(r0, GRP)]], bbufs[s], sems_b[s])
            return cpa, cpb

        pend = {0: issue(0)}
        out_pend = {}
        for g in range(NGRP):
            s = g % 2
            if g + 1 < NGRP:
                pend[g + 1] = issue(g + 1)
            cpa, cpb = pend.pop(g)
            cpa.wait()
            cpb.wait()
            if g >= 2:
                out_pend.pop(g - 2).wait()
            abuf, bbuf, obuf = abufs[s], bbufs[s], obufs[s]
            r0 = g * GRP
            c0v = cfv[0, pl.ds(r0, GRP)]
            c1v = cfv[1, pl.ds(r0, GRP)]
            c2v = cfv[2, pl.ds(r0, GRP)]
            c3v = cfv[3, pl.ds(r0, GRP)]
            for r in range(GRP):
                c0, c1, c2, c3 = c0v[r], c1v[r], c2v[r], c3v[r]

                def col_fn(j, carry2, r=r, c0=c0, c1=c1, c2=c2, c3=c3,
                           abuf=abuf, bbuf=bbuf, obuf=obuf):
                    sl = pl.ds(j * L, L)
                    av = abuf[r, sl]
                    bv = bbuf[r, sl]
                    obuf[r, sl] = (c0 + c1 * av) + (c2 + c3 * av) * bv
                    return carry2

                lax.fori_loop(0, BATCH // L, col_fn, 0)
            out_pend[g] = pltpu.async_copy(
                obuf, out.at[pl.ds(base + r0, GRP)], sems_o[s])
        for g in sorted(out_pend):
            out_pend.pop(g).wait()

    kfn = pl.kernel(
        body,
        out_type=jax.ShapeDtypeStruct((OUT_PAD, BATCH), jnp.float32),
        mesh=mesh,
        scratch_types=[
            pltpu.VMEM((BPW,), jnp.int32),
            pltpu.VMEM((BPW,), jnp.int32),
            pltpu.VMEM((4, BPW), jnp.float32),
            [pltpu.VMEM((GRP, BATCH), jnp.float32)] * 2,
            [pltpu.VMEM((GRP, BATCH), jnp.float32)] * 2,
            [pltpu.VMEM((GRP, BATCH), jnp.float32)] * 2,
            [pltpu.SemaphoreType.DMA] * 2,
            [pltpu.SemaphoreType.DMA] * 2,
            [pltpu.SemaphoreType.DMA] * 2,
        ],
    )
    return kfn(table, aidx, bidx, cfs)


def _gsum_tc(y, n_valid):
    rows = n_valid // NCLS

    def body(y_ref, o_ref):
        o_ref[...] = (jnp.sum(y_ref[...], axis=0, keepdims=True) / TAU)[None]

    return pl.pallas_call(
        body,
        grid=(NCLS,),
        in_specs=[pl.BlockSpec((rows, BATCH), lambda c: (c, 0))],
        out_specs=pl.BlockSpec((1, 1, BATCH), lambda c: (c, 0, 0)),
        out_shape=jax.ShapeDtypeStruct((NCLS, 1, BATCH), jnp.float32),
    )(y)


def kernel(x, w1, w2, w3, a1, b1, a2, b2, a3, b3):
    xt = x.T
    n3 = w3.shape[0]
    w3p = jnp.concatenate([w3, jnp.zeros((OUT_PAD - n3, 16), jnp.float32)], 0)
    wall = jnp.concatenate([w1, w2, w3p], axis=0)
    coefs = _coef_tc(wall).T
    cf1 = coefs[:, :OUT_PAD]
    cf2 = coefs[:, OUT_PAD:2 * OUT_PAD]
    cf3 = coefs[:, 2 * OUT_PAD:]
    padi = jnp.zeros((OUT_PAD - n3,), jnp.int32)
    a3p = jnp.concatenate([a3, padi])
    b3p = jnp.concatenate([b3, padi])
    y1 = _sc_layer(xt, a1, b1, cf1)
    y2 = _sc_layer(y1, a2, b2, cf2)
    y3 = _sc_layer(y2, a3p, b3p, cf3)
    cls = _gsum_tc(y3, n3)
    return cls.reshape(NCLS, BATCH).T


# speedup vs baseline: 1.9946x; 1.1337x over previous
import jax
import jax.numpy as jnp
from jax import lax
from jax.experimental import pallas as pl
from jax.experimental.pallas import tpu as pltpu
from jax.experimental.pallas import tpu_sc as plsc

BATCH = 1024
TAU = 30.0
NCLS = 10
NC, NS, L = 2, 16, 16
NW = NC * NS
OUT_PAD = 8192
BPW = OUT_PAD // NW
GRP = 16
NGRP = BPW // GRP
RQ = 4

_GATE_K = (
    (0, 0, 0, 0), (0, 0, 0, 1), (0, 1, 0, -1), (0, 1, 0, 0),
    (0, 0, 1, -1), (0, 0, 1, 0), (0, 1, 1, -2), (0, 1, 1, -1),
    (1, -1, -1, 1), (1, -1, -1, 2), (1, 0, -1, 0), (1, 0, -1, 1),
    (1, -1, 0, 0), (1, -1, 0, 1), (1, 0, 0, -1), (1, 0, 0, 0),
)


def _coef_tc(wall):
    def body(w_ref, k_ref, o_ref):
        w = w_ref[...]
        m = jnp.max(w, axis=-1, keepdims=True)
        e = jnp.exp(w - m)
        p = e / jnp.sum(e, axis=-1, keepdims=True)
        o_ref[...] = jax.lax.dot(p, k_ref[...], precision=lax.Precision.HIGHEST)

    n = wall.shape[0]
    blk = 2048
    return pl.pallas_call(
        body,
        grid=(n // blk,),
        in_specs=[
            pl.BlockSpec((blk, 16), lambda i: (i, 0)),
            pl.BlockSpec((16, 4), lambda i: (0, 0)),
        ],
        out_specs=pl.BlockSpec((blk, 4), lambda i: (i, 0)),
        out_shape=jax.ShapeDtypeStruct((n, 4), jnp.float32),
    )(wall, jnp.asarray(_GATE_K, dtype=jnp.float32))


def _sc_layer(table, aidx, bidx, cfs):
    mesh = plsc.VectorSubcoreMesh(core_axis_name="c", subcore_axis_name="s")

    def body(tab, ai, bi, cf, out, aiv, biv, cfv,
             abufs, bbufs, obufs, sems_a, sems_b, sems_o):
        wid = lax.axis_index("s") * NC + lax.axis_index("c")
        base = wid * BPW
        pltpu.sync_copy(ai.at[pl.ds(base, BPW)], aiv)
        pltpu.sync_copy(bi.at[pl.ds(base, BPW)], biv)
        pltpu.sync_copy(cf.at[:, pl.ds(base, BPW)], cfv)

        def issue(g):
            s = g % 2
            r0 = g * GRP
            cpa = pltpu.async_copy(
                tab.at[aiv.at[pl.ds(r0, GRP)]], abufs[s], sems_a[s])
            cpb = pltpu.async_copy(
                tab.at[biv.at[pl.ds(r0, GRP)]], bbufs[s], sems_b[s])
            return cpa, cpb

        pend = {0: issue(0)}
        out_pend = {}
        for g in range(NGRP):
            s = g % 2
            if g + 1 < NGRP:
                pend[g + 1] = issue(g + 1)
            cpa, cpb = pend.pop(g)
            cpa.wait()
            cpb.wait()
            if g >= 2:
                out_pend.pop(g - 2).wait()
            abuf, bbuf, obuf = abufs[s], bbufs[s], obufs[s]
            r0 = g * GRP
            c0v = cfv[0, pl.ds(r0, GRP)]
            c1v = cfv[1, pl.ds(r0, GRP)]
            c2v = cfv[2, pl.ds(r0, GRP)]
            c3v = cfv[3, pl.ds(r0, GRP)]
            for q in range(GRP // RQ):
                rows = [q * RQ + i for i in range(RQ)]
                cs = [(c0v[r], c1v[r], c2v[r], c3v[r]) for r in rows]

                def col_fn(j, carry2, rows=rows, cs=cs,
                           abuf=abuf, bbuf=bbuf, obuf=obuf):
                    sl = pl.ds(j * L, L)
                    for r, (c0, c1, c2, c3) in zip(rows, cs):
                        av = abuf[r, sl]
                        bv = bbuf[r, sl]
                        obuf[r, sl] = (c0 + c1 * av) + (c2 + c3 * av) * bv
                    return carry2

                lax.fori_loop(0, BATCH // L, col_fn, 0)
            out_pend[g] = pltpu.async_copy(
                obuf, out.at[pl.ds(base + r0, GRP)], sems_o[s])
        for g in sorted(out_pend):
            out_pend.pop(g).wait()

    kfn = pl.kernel(
        body,
        out_type=jax.ShapeDtypeStruct((OUT_PAD, BATCH), jnp.float32),
        mesh=mesh,
        scratch_types=[
            pltpu.VMEM((BPW,), jnp.int32),
            pltpu.VMEM((BPW,), jnp.int32),
            pltpu.VMEM((4, BPW), jnp.float32),
            [pltpu.VMEM((GRP, BATCH), jnp.float32)] * 2,
            [pltpu.VMEM((GRP, BATCH), jnp.float32)] * 2,
            [pltpu.VMEM((GRP, BATCH), jnp.float32)] * 2,
            [pltpu.SemaphoreType.DMA] * 2,
            [pltpu.SemaphoreType.DMA] * 2,
            [pltpu.SemaphoreType.DMA] * 2,
        ],
    )
    return kfn(table, aidx, bidx, cfs)


def _gsum_tc(y, n_valid):
    rows = n_valid // NCLS

    def body(y_ref, o_ref):
        o_ref[...] = (jnp.sum(y_ref[...], axis=0, keepdims=True) / TAU)[None]

    return pl.pallas_call(
        body,
        grid=(NCLS,),
        in_specs=[pl.BlockSpec((rows, BATCH), lambda c: (c, 0))],
        out_specs=pl.BlockSpec((1, 1, BATCH), lambda c: (c, 0, 0)),
        out_shape=jax.ShapeDtypeStruct((NCLS, 1, BATCH), jnp.float32),
    )(y)


def kernel(x, w1, w2, w3, a1, b1, a2, b2, a3, b3):
    xt = x.T
    n3 = w3.shape[0]
    w3p = jnp.concatenate([w3, jnp.zeros((OUT_PAD - n3, 16), jnp.float32)], 0)
    wall = jnp.concatenate([w1, w2, w3p], axis=0)
    coefs = _coef_tc(wall).T
    cf1 = coefs[:, :OUT_PAD]
    cf2 = coefs[:, OUT_PAD:2 * OUT_PAD]
    cf3 = coefs[:, 2 * OUT_PAD:]
    padi = jnp.zeros((OUT_PAD - n3,), jnp.int32)
    a3p = jnp.concatenate([a3, padi])
    b3p = jnp.concatenate([b3, padi])
    y1 = _sc_layer(xt, a1, b1, cf1)
    y2 = _sc_layer(y1, a2, b2, cf2)
    y3 = _sc_layer(y2, a3p, b3p, cf3)
    cls = _gsum_tc(y3, n3)
    return cls.reshape(NCLS, BATCH).T
